# baseline (speedup 1.0000x reference)
import jax
import jax.numpy as jnp
from jax.experimental import pallas as pl
from jax.experimental.pallas import tpu as pltpu

_BM = 200


def _gcn_kernel(seq_ref, wt_ref, adja_ref, adjb_ref, bias_ref, a_ref,
                o_ref, fts_ref):
    @pl.when(pl.program_id(0) == 0)
    def _():
        fts_ref[...] = jnp.dot(
            seq_ref[...], wt_ref[...], preferred_element_type=jnp.float32
        )

    a = a_ref[0, 0]
    outa = jnp.dot(
        adja_ref[...], fts_ref[...], preferred_element_type=jnp.float32
    ) + bias_ref[...]
    o_ref[0:_BM, :] = jnp.where(outa > 0, outa, a * outa)
    outb = jnp.dot(
        adjb_ref[...], fts_ref[...], preferred_element_type=jnp.float32
    ) + bias_ref[...]
    o_ref[_BM:, :] = jnp.where(outb > 0, outb, a * outb)


def kernel(seq, adj, W, bias, prelu_a):
    n, d_in = seq.shape
    d_out = W.shape[0]

    out = pl.pallas_call(
        _gcn_kernel,
        grid=(n // (2 * _BM),),
        in_specs=[
            pl.BlockSpec((n, d_in), lambda i: (0, 0)),
            pl.BlockSpec((d_in, d_out), lambda i: (0, 0)),
            pl.BlockSpec((_BM, n), lambda i: (2 * i, 0)),
            pl.BlockSpec((_BM, n), lambda i: (2 * i + 1, 0)),
            pl.BlockSpec((1, d_out), lambda i: (0, 0)),
            pl.BlockSpec((1, 1), lambda i: (0, 0)),
        ],
        out_specs=pl.BlockSpec((2 * _BM, d_out), lambda i: (i, 0)),
        out_shape=jax.ShapeDtypeStruct((n, d_out), jnp.float32),
        scratch_shapes=[pltpu.VMEM((n, d_out), jnp.float32)],
        compiler_params=pltpu.CompilerParams(
            dimension_semantics=("arbitrary",),
            vmem_limit_bytes=62 * 1024 * 1024,
        ),
    )(seq, W.T, adj, adj, bias.reshape(1, d_out), prelu_a.reshape(1, 1))
    return out
